# baseline (speedup 1.0000x reference)
import jax
import jax.numpy as jnp
from jax.experimental import pallas as pl
from jax.experimental.pallas import tpu as pltpu

_EPS = 1e-5


def _conv_stats_kernel(x_ref, w_ref, y_ref, s_ref):
    bn, cin, h, w = x_ref.shape
    wci = w_ref.shape[1]
    wco = w_ref.shape[2]

    xb = x_ref[...].astype(jnp.bfloat16)
    xt = jnp.transpose(xb, (0, 1, 3, 2)).reshape(bn, wci, h)
    xr = jnp.transpose(xt, (0, 2, 1))

    zrow = jnp.zeros((bn, 1, wci), jnp.bfloat16)
    a0 = jnp.concatenate([zrow, xr[:, : h - 1, :]], axis=1).reshape(bn * h, wci)
    a2 = jnp.concatenate([xr[:, 1:, :], zrow], axis=1).reshape(bn * h, wci)
    a1 = xr.reshape(bn * h, wci)

    acc = jnp.dot(a0, w_ref[0], preferred_element_type=jnp.float32)
    acc = acc + jnp.dot(a1, w_ref[1], preferred_element_type=jnp.float32)
    acc = acc + jnp.dot(a2, w_ref[2], preferred_element_type=jnp.float32)

    s_ref[0, 0:1, :] = jnp.sum(acc, axis=0, keepdims=True)
    s_ref[0, 1:2, :] = jnp.sum(acc * acc, axis=0, keepdims=True)

    y_ref[...] = acc.astype(jnp.bfloat16).reshape(bn, h, wco)


def _bn_relu_kernel(y_ref, sc_ref, sh_ref, o_ref):
    bn, cout, h, w = o_ref.shape
    y = y_ref[...].astype(jnp.float32)
    z = jnp.maximum(y * sc_ref[...] + sh_ref[...], 0.0)
    zt = jnp.transpose(z, (0, 2, 1)).reshape(bn, cout, w, h)
    o_ref[...] = jnp.transpose(zt, (0, 1, 3, 2))


def kernel(x_nchw, w_oihw, bias, gamma, beta):
    N, Cin, H, W = x_nchw.shape
    Cout = w_oihw.shape[0]
    WCI, WCO = W * Cin, W * Cout
    f32 = jnp.float32
    bf16 = jnp.bfloat16

    BN = 32 if N % 32 == 0 else 1
    GN = N // BN

    wt = jnp.transpose(w_oihw, (2, 3, 1, 0)).astype(f32)
    sel = jnp.stack([jnp.eye(W, W, k=1 - kw, dtype=f32) for kw in range(3)])
    wband = jnp.einsum("kab,hkio->hiaob", sel, wt).reshape(3, WCI, WCO)
    wband = wband.astype(bf16)

    y_rows, stats = pl.pallas_call(
        _conv_stats_kernel,
        out_shape=[
            jax.ShapeDtypeStruct((N, H, WCO), bf16),
            jax.ShapeDtypeStruct((GN, 2, WCO), f32),
        ],
        grid=(GN,),
        in_specs=[
            pl.BlockSpec((BN, Cin, H, W), lambda i: (i, 0, 0, 0)),
            pl.BlockSpec((3, WCI, WCO), lambda i: (0, 0, 0)),
        ],
        out_specs=[
            pl.BlockSpec((BN, H, WCO), lambda i: (i, 0, 0)),
            pl.BlockSpec((1, 2, WCO), lambda i: (i, 0, 0)),
        ],
        compiler_params=pltpu.CompilerParams(
            dimension_semantics=("parallel",),
        ),
    )(x_nchw, wband)

    tot = jnp.sum(stats, axis=0).reshape(2, Cout, W).sum(axis=2)
    m = float(N * H * W)
    mean = tot[0] / m
    var = jnp.maximum(tot[1] / m - mean * mean, 0.0)
    scale = gamma.astype(f32) * jax.lax.rsqrt(var + _EPS)
    shift = beta.astype(f32) - mean * scale
    scale_row = jnp.repeat(scale, W).reshape(1, WCO)
    shift_row = jnp.repeat(shift, W).reshape(1, WCO)

    out = pl.pallas_call(
        _bn_relu_kernel,
        out_shape=jax.ShapeDtypeStruct((N, Cout, H, W), f32),
        grid=(GN,),
        in_specs=[
            pl.BlockSpec((BN, H, WCO), lambda i: (i, 0, 0)),
            pl.BlockSpec((1, WCO), lambda i: (0, 0)),
            pl.BlockSpec((1, WCO), lambda i: (0, 0)),
        ],
        out_specs=pl.BlockSpec((BN, Cout, H, W), lambda i: (i, 0, 0, 0)),
        compiler_params=pltpu.CompilerParams(
            dimension_semantics=("parallel",),
        ),
    )(y_rows, scale_row, shift_row)

    return out


# speedup vs baseline: 1.5182x; 1.5182x over previous
import jax
import jax.numpy as jnp
from jax.experimental import pallas as pl
from jax.experimental.pallas import tpu as pltpu

_EPS = 1e-5


def _conv_stats_kernel(x_ref, w_ref, y_ref, s_ref):
    bn, hh, wco = y_ref.shape
    wci = x_ref.shape[2]
    xr = x_ref[...].astype(jnp.bfloat16)
    zrow = jnp.zeros((bn, 1, wci), jnp.bfloat16)
    a0 = jnp.concatenate([zrow, xr[:, : hh - 1, :]], axis=1).reshape(bn * hh, wci)
    a2 = jnp.concatenate([xr[:, 1:, :], zrow], axis=1).reshape(bn * hh, wci)
    a1 = xr.reshape(bn * hh, wci)
    acc = jnp.dot(a0, w_ref[0], preferred_element_type=jnp.float32)
    acc = acc + jnp.dot(a1, w_ref[1], preferred_element_type=jnp.float32)
    acc = acc + jnp.dot(a2, w_ref[2], preferred_element_type=jnp.float32)
    s_ref[0, 0:1, :] = jnp.sum(acc, axis=0, keepdims=True)
    s_ref[0, 1:2, :] = jnp.sum(acc * acc, axis=0, keepdims=True)
    y_ref[...] = acc.reshape(bn, hh, wco).astype(jnp.bfloat16)


def _bn_relu_kernel(y_ref, sc_ref, sh_ref, o_ref):
    y = y_ref[...].astype(jnp.float32)
    o_ref[...] = jnp.maximum(y * sc_ref[...] + sh_ref[...], 0.0)


def kernel(x_nchw, w_oihw, bias, gamma, beta):
    N, Cin, H, W = x_nchw.shape
    Cout = w_oihw.shape[0]
    WCI, WCO = W * Cin, W * Cout
    f32 = jnp.float32
    bf16 = jnp.bfloat16

    BN = 32 if N % 32 == 0 else 1
    GN = N // BN

    x_flat = jnp.transpose(x_nchw, (0, 2, 3, 1)).reshape(N, H, WCI)

    wt = jnp.transpose(w_oihw, (2, 3, 1, 0)).astype(f32)
    sel = jnp.stack([jnp.eye(W, W, k=1 - kw, dtype=f32) for kw in range(3)])
    wband = jnp.einsum("kab,hkio->haibo", sel, wt).reshape(3, WCI, WCO)
    wband = wband.astype(bf16)

    y_slab, stats = pl.pallas_call(
        _conv_stats_kernel,
        out_shape=[
            jax.ShapeDtypeStruct((N, H, WCO), bf16),
            jax.ShapeDtypeStruct((GN, 2, WCO), f32),
        ],
        grid=(GN,),
        in_specs=[
            pl.BlockSpec((BN, H, WCI), lambda i: (i, 0, 0)),
            pl.BlockSpec((3, WCI, WCO), lambda i: (0, 0, 0)),
        ],
        out_specs=[
            pl.BlockSpec((BN, H, WCO), lambda i: (i, 0, 0)),
            pl.BlockSpec((1, 2, WCO), lambda i: (i, 0, 0)),
        ],
        compiler_params=pltpu.CompilerParams(
            dimension_semantics=("parallel",),
        ),
    )(x_flat, wband)

    tot = jnp.sum(stats, axis=0).reshape(2, W, Cout).sum(axis=1)
    m = float(N * H * W)
    mean = tot[0] / m
    var = jnp.maximum(tot[1] / m - mean * mean, 0.0)
    scale = gamma.astype(f32) * jax.lax.rsqrt(var + _EPS)
    shift = beta.astype(f32) - mean * scale
    scale_row = jnp.tile(scale, W).reshape(1, WCO)
    shift_row = jnp.tile(shift, W).reshape(1, WCO)

    out_slab = pl.pallas_call(
        _bn_relu_kernel,
        out_shape=jax.ShapeDtypeStruct((N, H, WCO), f32),
        grid=(GN,),
        in_specs=[
            pl.BlockSpec((BN, H, WCO), lambda i: (i, 0, 0)),
            pl.BlockSpec((1, WCO), lambda i: (0, 0)),
            pl.BlockSpec((1, WCO), lambda i: (0, 0)),
        ],
        out_specs=pl.BlockSpec((BN, H, WCO), lambda i: (i, 0, 0)),
        compiler_params=pltpu.CompilerParams(
            dimension_semantics=("parallel",),
        ),
    )(y_slab, scale_row, shift_row)

    out = out_slab.reshape(N, H, W, Cout)
    return jnp.transpose(out, (0, 3, 1, 2))
